# Initial kernel scaffold; baseline (speedup 1.0000x reference)
import jax
import jax.numpy as jnp
from jax.experimental import pallas as pl


def kernel(x, edge_index, W, b, weights):
    raise NotImplementedError("write your pallas kernel here")


# speedup vs baseline: 3.4149x; 3.4149x over previous
import functools

import jax
import jax.numpy as jnp
from jax import lax
from jax.experimental import pallas as pl
from jax.experimental.pallas import tpu as pltpu
from jax.experimental.pallas import tpu_sc as plsc

N0 = 10000
E0 = 160000
D = 256
H = 128
NP = 10240
TRASH = N0
EPAD = 163840
ROWS_PER_TILE = NP // 16
EDGE_ROWS = EPAD // 128

_mesh = plsc.VectorSubcoreMesh(core_axis_name="c", subcore_axis_name="s")


@functools.partial(
    pl.kernel,
    mesh=_mesh,
    out_type=[
        jax.ShapeDtypeStruct((NP, H), jnp.float32),
        jax.ShapeDtypeStruct((NP, H), jnp.float32),
    ],
    scratch_types=[
        pltpu.VMEM((80, 128), jnp.int32),
        pltpu.VMEM((80, 128), jnp.int32),
        pltpu.VMEM((128, H), jnp.float32),
        pltpu.VMEM((64, H), jnp.float32),
        pltpu.VMEM_SHARED((NP, H), jnp.float32),
        pltpu.SemaphoreType.DMA,
    ],
)
def _adj_kernel(w0_hbm, w1_hbm, col2_hbm, row2_hbm, a0_hbm, a1_hbm,
                colv, rowv, gbuf, tb, acc, sem):
    cid = lax.axis_index("c")
    sid = lax.axis_index("s")

    def _zrow(i, _):
        def _zc(k, _2):
            tb[i, pl.ds(k * 16, 16)] = jnp.zeros((16,), jnp.float32)
            return 0

        lax.fori_loop(0, 8, _zc, 0)
        return 0

    lax.fori_loop(0, 64, _zrow, 0)

    def _zacc(t, _):
        pltpu.sync_copy(tb, acc.at[pl.ds(sid * ROWS_PER_TILE + t * 64, 64)])
        return 0

    lax.fori_loop(0, 10, _zacc, 0)
    plsc.subcore_barrier()

    pltpu.sync_copy(col2_hbm.at[pl.ds(sid * 80, 80)], colv)
    pltpu.sync_copy(row2_hbm.at[pl.ds(sid * 80, 80)], rowv)

    def _step(j, _):
        @pl.when(cid == 0)
        def _():
            pltpu.async_copy(w0_hbm.at[colv.at[j]], gbuf, sem).wait()

        @pl.when(cid == 1)
        def _():
            pltpu.async_copy(w1_hbm.at[colv.at[j]], gbuf, sem).wait()

        pltpu.sync_copy(gbuf, acc.at[rowv.at[j]], add=True)
        return 0

    lax.fori_loop(0, 80, _step, 0)
    plsc.subcore_barrier()

    def _wout(t, _):
        r0 = sid * ROWS_PER_TILE + t * 64
        pltpu.sync_copy(acc.at[pl.ds(r0, 64)], tb)

        @pl.when(cid == 0)
        def _():
            pltpu.sync_copy(tb, a0_hbm.at[pl.ds(r0, 64)])

        @pl.when(cid == 1)
        def _():
            pltpu.sync_copy(tb, a1_hbm.at[pl.ds(r0, 64)])

        return 0

    lax.fori_loop(0, 10, _wout, 0)


def _mm_body(x_ref, wt_ref, b_ref, o_ref):
    o_ref[...] = (
        jnp.dot(x_ref[...], wt_ref[...], preferred_element_type=jnp.float32)
        + b_ref[...]
    )


def _matmul(xp, wt, b2):
    return pl.pallas_call(
        _mm_body,
        grid=(NP // 256,),
        in_specs=[
            pl.BlockSpec((256, D), lambda i: (i, 0)),
            pl.BlockSpec((D, D), lambda i: (0, 0)),
            pl.BlockSpec((1, D), lambda i: (0, 0)),
        ],
        out_specs=pl.BlockSpec((256, D), lambda i: (i, 0)),
        out_shape=jax.ShapeDtypeStruct((NP, D), jnp.float32),
    )(xp, wt, b2)


def _dinv_from(deg_ref):
    deg = deg_ref[:, 0:1]
    return jnp.where(deg > 0.0, lax.rsqrt(jnp.maximum(deg, 1e-30)), 0.0)


def _scale0_body(deg_ref, z_ref, w0_ref, w1_ref):
    dinv = _dinv_from(deg_ref)
    w0_ref[...] = z_ref[:, :H] * dinv
    w1_ref[...] = z_ref[:, H:] * dinv


def _scale0(deg, z):
    return pl.pallas_call(
        _scale0_body,
        grid=(NP // 512,),
        in_specs=[
            pl.BlockSpec((512, H), lambda i: (i, 0)),
            pl.BlockSpec((512, D), lambda i: (i, 0)),
        ],
        out_specs=[
            pl.BlockSpec((512, H), lambda i: (i, 0)),
            pl.BlockSpec((512, H), lambda i: (i, 0)),
        ],
        out_shape=[
            jax.ShapeDtypeStruct((NP, H), jnp.float32),
            jax.ShapeDtypeStruct((NP, H), jnp.float32),
        ],
    )(deg, z)


def _combine_body(c_ref, deg_ref, a0_ref, a1_ref, o_ref,
                  out_ref, wn0_ref, wn1_ref):
    dinv = _dinv_from(deg_ref)
    c = c_ref[0]
    t0 = a0_ref[...] * dinv
    t1 = a1_ref[...] * dinv
    out_ref[:, :H] = o_ref[:, :H] + c * t0
    out_ref[:, H:] = o_ref[:, H:] + c * t1
    wn0_ref[...] = t0 * dinv
    wn1_ref[...] = t1 * dinv


def _combine(coef, deg, a0, a1, o):
    return pl.pallas_call(
        _combine_body,
        grid=(NP // 512,),
        in_specs=[
            pl.BlockSpec(memory_space=pltpu.SMEM),
            pl.BlockSpec((512, H), lambda i: (i, 0)),
            pl.BlockSpec((512, H), lambda i: (i, 0)),
            pl.BlockSpec((512, H), lambda i: (i, 0)),
            pl.BlockSpec((512, D), lambda i: (i, 0)),
        ],
        out_specs=[
            pl.BlockSpec((512, D), lambda i: (i, 0)),
            pl.BlockSpec((512, H), lambda i: (i, 0)),
            pl.BlockSpec((512, H), lambda i: (i, 0)),
        ],
        out_shape=[
            jax.ShapeDtypeStruct((NP, D), jnp.float32),
            jax.ShapeDtypeStruct((NP, H), jnp.float32),
            jax.ShapeDtypeStruct((NP, H), jnp.float32),
        ],
    )(coef, deg, a0, a1, o)


def kernel(x, edge_index, W, b, weights):
    row = edge_index[0].astype(jnp.int32)
    col = edge_index[1].astype(jnp.int32)
    pad = jnp.full((EPAD - E0,), TRASH, jnp.int32)
    row2 = jnp.concatenate([row, pad]).reshape(EDGE_ROWS, 128)
    col2 = jnp.concatenate([col, pad]).reshape(EDGE_ROWS, 128)
    xp = jnp.pad(x, ((0, NP - N0), (0, 0)))
    wt = W.T
    b2 = b.reshape(1, D)

    ones = jnp.ones((NP, H), jnp.float32)
    z = _matmul(xp, wt, b2)
    deg, _ = _adj_kernel(ones, ones, col2, row2)
    w0, w1 = _scale0(deg, z)

    c1 = weights[0].reshape(1)
    c2 = weights[1].reshape(1)
    c3 = (weights[0] * weights[1]).reshape(1)

    a0, a1 = _adj_kernel(w0, w1, col2, row2)
    o, w0, w1 = _combine(c1, deg, a0, a1, z)
    a0, a1 = _adj_kernel(w0, w1, col2, row2)
    o, w0, w1 = _combine(c2, deg, a0, a1, o)
    a0, a1 = _adj_kernel(w0, w1, col2, row2)
    o, _, _ = _combine(c3, deg, a0, a1, o)
    return o[:N0]


# speedup vs baseline: 3.5972x; 1.0534x over previous
import functools

import jax
import jax.numpy as jnp
from jax import lax
from jax.experimental import pallas as pl
from jax.experimental.pallas import tpu as pltpu
from jax.experimental.pallas import tpu_sc as plsc

N0 = 10000
E0 = 160000
D = 256
H = 128
NP = 10240
TRASH = N0
EPAD = 163840
C = 128
EROWS = EPAD // C
CPS = EROWS // 16
CPT = EROWS // 32
QC = 16
RPT = NP // 16

_mesh = plsc.VectorSubcoreMesh(core_axis_name="c", subcore_axis_name="s")


@functools.partial(
    pl.kernel,
    mesh=_mesh,
    out_type=[
        jax.ShapeDtypeStruct((NP, 16), jnp.float32),
        jax.ShapeDtypeStruct((NP, 16), jnp.float32),
    ],
    scratch_types=[
        pltpu.VMEM((CPT, C), jnp.int32),
        pltpu.VMEM((C, 16), jnp.float32),
        pltpu.VMEM((C, 16), jnp.float32),
        pltpu.VMEM_SHARED((NP, 16), jnp.float32),
        pltpu.SemaphoreType.DMA,
    ],
)
def _deg_kernel(row2_hbm, deg0_hbm, deg1_hbm, rowv, ones_v, zb, acc, sem):
    cid = lax.axis_index("c")
    sid = lax.axis_index("s")
    wid = cid * 16 + sid

    def _fill(i, _):
        ones_v[i, :] = jnp.ones((16,), jnp.float32)
        zb[i, :] = jnp.zeros((16,), jnp.float32)
        return 0

    lax.fori_loop(0, C, _fill, 0)

    def _zacc(t, _):
        pltpu.sync_copy(zb, acc.at[pl.ds(sid * RPT + t * C, C)])
        return 0

    lax.fori_loop(0, RPT // C, _zacc, 0)
    plsc.subcore_barrier()

    pltpu.sync_copy(row2_hbm.at[pl.ds(wid * CPT, CPT)], rowv)

    def _step(g, _):
        c0 = 4 * g
        d0 = pltpu.async_copy(ones_v, acc.at[rowv.at[c0]], sem, add=True)
        d1 = pltpu.async_copy(ones_v, acc.at[rowv.at[c0 + 1]], sem, add=True)
        d2 = pltpu.async_copy(ones_v, acc.at[rowv.at[c0 + 2]], sem, add=True)
        d3 = pltpu.async_copy(ones_v, acc.at[rowv.at[c0 + 3]], sem, add=True)
        d0.wait()
        d1.wait()
        d2.wait()
        d3.wait()
        return 0

    lax.fori_loop(0, CPT // 4, _step, 0)
    plsc.subcore_barrier()

    def _wout(t, _):
        r0 = sid * RPT + t * C
        pltpu.sync_copy(acc.at[pl.ds(r0, C)], zb)

        @pl.when(cid == 0)
        def _():
            pltpu.sync_copy(zb, deg0_hbm.at[pl.ds(r0, C)])

        @pl.when(cid == 1)
        def _():
            pltpu.sync_copy(zb, deg1_hbm.at[pl.ds(r0, C)])

        return 0

    lax.fori_loop(0, RPT // C, _wout, 0)


@functools.partial(
    pl.kernel,
    mesh=_mesh,
    out_type=[
        jax.ShapeDtypeStruct((NP, H), jnp.float32),
        jax.ShapeDtypeStruct((NP, H), jnp.float32),
    ],
    scratch_types=[
        pltpu.VMEM((QC, C), jnp.int32),
        pltpu.VMEM((QC, C), jnp.int32),
        pltpu.VMEM((C, H), jnp.float32),
        pltpu.VMEM((C, H), jnp.float32),
        pltpu.VMEM_SHARED((NP, H), jnp.float32),
        pltpu.SemaphoreType.DMA,
        pltpu.SemaphoreType.DMA,
        pltpu.SemaphoreType.DMA,
        pltpu.SemaphoreType.DMA,
    ],
)
def _adj_kernel(w0_hbm, w1_hbm, col2_hbm, row2_hbm, a0_hbm, a1_hbm,
                colv, rowv, bufa, bufb, acc, ga, gb, sa, sb):
    cid = lax.axis_index("c")
    sid = lax.axis_index("s")

    def _gather(c, buf, sem):
        @pl.when(cid == 0)
        def _():
            pltpu.async_copy(w0_hbm.at[colv.at[c]], buf, sem)

        @pl.when(cid == 1)
        def _():
            pltpu.async_copy(w1_hbm.at[colv.at[c]], buf, sem)

    def _gwait(c, buf, sem):
        pltpu.make_async_copy(w0_hbm.at[colv.at[c]], buf, sem).wait()

    def _swait(c, buf, sem):
        pltpu.make_async_copy(buf, acc.at[rowv.at[c]], sem).wait()

    def _zrow(i, _):
        def _zc(k, _2):
            bufa[i, pl.ds(k * 16, 16)] = jnp.zeros((16,), jnp.float32)
            return 0

        lax.fori_loop(0, 8, _zc, 0)
        return 0

    lax.fori_loop(0, C, _zrow, 0)

    def _zacc(t, _):
        pltpu.sync_copy(bufa, acc.at[pl.ds(sid * RPT + t * C, C)])
        return 0

    lax.fori_loop(0, RPT // C, _zacc, 0)
    plsc.subcore_barrier()

    def _quarter(q, _):
        pltpu.sync_copy(col2_hbm.at[pl.ds(sid * CPS + q * QC, QC)], colv)
        pltpu.sync_copy(row2_hbm.at[pl.ds(sid * CPS + q * QC, QC)], rowv)
        _gather(0, bufa, ga)
        _gather(1, bufb, gb)

        def _step(jj, _):
            c0 = 2 * jj
            c1 = c0 + 1
            _gwait(c0, bufa, ga)
            pltpu.async_copy(bufa, acc.at[rowv.at[c0]], sa, add=True)
            _gwait(c1, bufb, gb)
            pltpu.async_copy(bufb, acc.at[rowv.at[c1]], sb, add=True)

            @pl.when(jj < QC // 2 - 1)
            def _():
                _swait(c0, bufa, sa)
                _gather(c0 + 2, bufa, ga)
                _swait(c1, bufb, sb)
                _gather(c1 + 2, bufb, gb)

            return 0

        lax.fori_loop(0, QC // 2, _step, 0)
        _swait(QC - 2, bufa, sa)
        _swait(QC - 1, bufb, sb)
        return 0

    lax.fori_loop(0, CPS // QC, _quarter, 0)
    plsc.subcore_barrier()

    def _wout(t, _):
        r0 = sid * RPT + t * C
        pltpu.sync_copy(acc.at[pl.ds(r0, C)], bufa)

        @pl.when(cid == 0)
        def _():
            pltpu.sync_copy(bufa, a0_hbm.at[pl.ds(r0, C)])

        @pl.when(cid == 1)
        def _():
            pltpu.sync_copy(bufa, a1_hbm.at[pl.ds(r0, C)])

        return 0

    lax.fori_loop(0, RPT // C, _wout, 0)


def _mm_body(x_ref, wt_ref, b_ref, o_ref):
    o_ref[...] = (
        jnp.dot(x_ref[...], wt_ref[...], preferred_element_type=jnp.float32)
        + b_ref[...]
    )


def _matmul(xp, wt, b2):
    return pl.pallas_call(
        _mm_body,
        grid=(NP // 256,),
        in_specs=[
            pl.BlockSpec((256, D), lambda i: (i, 0)),
            pl.BlockSpec((D, D), lambda i: (0, 0)),
            pl.BlockSpec((1, D), lambda i: (0, 0)),
        ],
        out_specs=pl.BlockSpec((256, D), lambda i: (i, 0)),
        out_shape=jax.ShapeDtypeStruct((NP, D), jnp.float32),
    )(xp, wt, b2)


def _dinv_from(deg0_ref, deg1_ref):
    deg = deg0_ref[:, 0:1] + deg1_ref[:, 0:1]
    return jnp.where(deg > 0.0, lax.rsqrt(jnp.maximum(deg, 1e-30)), 0.0)


def _scale0_body(deg0_ref, deg1_ref, z_ref, w0_ref, w1_ref):
    dinv = _dinv_from(deg0_ref, deg1_ref)
    w0_ref[...] = z_ref[:, :H] * dinv
    w1_ref[...] = z_ref[:, H:] * dinv


def _scale0(deg0, deg1, z):
    return pl.pallas_call(
        _scale0_body,
        grid=(NP // 512,),
        in_specs=[
            pl.BlockSpec((512, 16), lambda i: (i, 0)),
            pl.BlockSpec((512, 16), lambda i: (i, 0)),
            pl.BlockSpec((512, D), lambda i: (i, 0)),
        ],
        out_specs=[
            pl.BlockSpec((512, H), lambda i: (i, 0)),
            pl.BlockSpec((512, H), lambda i: (i, 0)),
        ],
        out_shape=[
            jax.ShapeDtypeStruct((NP, H), jnp.float32),
            jax.ShapeDtypeStruct((NP, H), jnp.float32),
        ],
    )(deg0, deg1, z)


def _combine_body(c_ref, deg0_ref, deg1_ref, a0_ref, a1_ref, o_ref,
                  out_ref, wn0_ref, wn1_ref):
    dinv = _dinv_from(deg0_ref, deg1_ref)
    c = c_ref[0]
    t0 = a0_ref[...] * dinv
    t1 = a1_ref[...] * dinv
    out_ref[:, :H] = o_ref[:, :H] + c * t0
    out_ref[:, H:] = o_ref[:, H:] + c * t1
    wn0_ref[...] = t0 * dinv
    wn1_ref[...] = t1 * dinv


def _combine(coef, deg0, deg1, a0, a1, o):
    return pl.pallas_call(
        _combine_body,
        grid=(NP // 512,),
        in_specs=[
            pl.BlockSpec(memory_space=pltpu.SMEM),
            pl.BlockSpec((512, 16), lambda i: (i, 0)),
            pl.BlockSpec((512, 16), lambda i: (i, 0)),
            pl.BlockSpec((512, H), lambda i: (i, 0)),
            pl.BlockSpec((512, H), lambda i: (i, 0)),
            pl.BlockSpec((512, D), lambda i: (i, 0)),
        ],
        out_specs=[
            pl.BlockSpec((512, D), lambda i: (i, 0)),
            pl.BlockSpec((512, H), lambda i: (i, 0)),
            pl.BlockSpec((512, H), lambda i: (i, 0)),
        ],
        out_shape=[
            jax.ShapeDtypeStruct((NP, D), jnp.float32),
            jax.ShapeDtypeStruct((NP, H), jnp.float32),
            jax.ShapeDtypeStruct((NP, H), jnp.float32),
        ],
    )(coef, deg0, deg1, a0, a1, o)


def kernel(x, edge_index, W, b, weights):
    row = edge_index[0].astype(jnp.int32)
    col = edge_index[1].astype(jnp.int32)
    pad = jnp.full((EPAD - E0,), TRASH, jnp.int32)
    row2 = jnp.concatenate([row, pad]).reshape(EROWS, C)
    col2 = jnp.concatenate([col, pad]).reshape(EROWS, C)
    xp = jnp.pad(x, ((0, NP - N0), (0, 0)))
    wt = W.T
    b2 = b.reshape(1, D)

    z = _matmul(xp, wt, b2)
    ones = jnp.ones((NP, H), jnp.float32)
    deg0, _unused = _adj_kernel(ones, ones, col2, row2)
    deg0 = deg0[:, :16]
    deg1 = jnp.zeros((NP, 16), jnp.float32)
    w0, w1 = _scale0(deg0, deg1, z)

    c1 = weights[0].reshape(1)
    c2 = weights[1].reshape(1)
    c3 = (weights[0] * weights[1]).reshape(1)

    a0, a1 = _adj_kernel(w0, w1, col2, row2)
    o, w0, w1 = _combine(c1, deg0, deg1, a0, a1, z)
    a0, a1 = _adj_kernel(w0, w1, col2, row2)
    o, w0, w1 = _combine(c2, deg0, deg1, a0, a1, o)
    a0, a1 = _adj_kernel(w0, w1, col2, row2)
    o, _, _ = _combine(c3, deg0, deg1, a0, a1, o)
    return o[:N0]


# speedup vs baseline: 4.8161x; 1.3388x over previous
import functools

import jax
import jax.numpy as jnp
from jax import lax
from jax.experimental import pallas as pl
from jax.experimental.pallas import tpu as pltpu
from jax.experimental.pallas import tpu_sc as plsc

N0 = 10000
E0 = 160000
D = 256
H = 128
NP = 10240
TRASH = N0
EPAD = 163840
C = 128
EROWS = EPAD // C
CPS = EROWS // 16
CPT = EROWS // 32
QC = 16
RPT = NP // 16

_mesh = plsc.VectorSubcoreMesh(core_axis_name="c", subcore_axis_name="s")


@functools.partial(
    pl.kernel,
    mesh=_mesh,
    out_type=[
        jax.ShapeDtypeStruct((NP, H), jnp.float32),
        jax.ShapeDtypeStruct((NP, H), jnp.float32),
    ],
    scratch_types=[
        pltpu.VMEM((CPT, C), jnp.int32),
        pltpu.VMEM((C, H), jnp.float32),
        pltpu.VMEM_SHARED((NP, H), jnp.float32),
        pltpu.SemaphoreType.DMA,
    ],
)
def _deg_kernel(row2_hbm, deg0_hbm, deg1_hbm, rowv, ones_v, acc, sem):
    cid = lax.axis_index("c")
    sid = lax.axis_index("s")
    wid = cid * 16 + sid

    def _zrow(i, _):
        def _zc(k, _2):
            ones_v[i, pl.ds(k * 16, 16)] = jnp.zeros((16,), jnp.float32)
            return 0

        lax.fori_loop(0, 8, _zc, 0)
        return 0

    lax.fori_loop(0, C, _zrow, 0)

    def _zacc(t, _):
        pltpu.sync_copy(ones_v, acc.at[pl.ds(sid * RPT + t * C, C)])
        return 0

    lax.fori_loop(0, RPT // C, _zacc, 0)

    def _frow(i, _):
        def _fc(k, _2):
            ones_v[i, pl.ds(k * 16, 16)] = jnp.ones((16,), jnp.float32)
            return 0

        lax.fori_loop(0, 8, _fc, 0)
        return 0

    lax.fori_loop(0, C, _frow, 0)
    plsc.subcore_barrier()

    pltpu.sync_copy(row2_hbm.at[pl.ds(wid * CPT, CPT)], rowv)

    def _step(g, _):
        c0 = 4 * g
        d0 = pltpu.async_copy(ones_v, acc.at[rowv.at[c0]], sem, add=True)
        d1 = pltpu.async_copy(ones_v, acc.at[rowv.at[c0 + 1]], sem, add=True)
        d2 = pltpu.async_copy(ones_v, acc.at[rowv.at[c0 + 2]], sem, add=True)
        d3 = pltpu.async_copy(ones_v, acc.at[rowv.at[c0 + 3]], sem, add=True)
        d0.wait()
        d1.wait()
        d2.wait()
        d3.wait()
        return 0

    lax.fori_loop(0, CPT // 4, _step, 0)
    plsc.subcore_barrier()

    def _wout(t, _):
        r0 = sid * RPT + t * C
        pltpu.sync_copy(acc.at[pl.ds(r0, C)], ones_v)

        @pl.when(cid == 0)
        def _():
            pltpu.sync_copy(ones_v, deg0_hbm.at[pl.ds(r0, C)])

        @pl.when(cid == 1)
        def _():
            pltpu.sync_copy(ones_v, deg1_hbm.at[pl.ds(r0, C)])

        return 0

    lax.fori_loop(0, RPT // C, _wout, 0)


@functools.partial(
    pl.kernel,
    mesh=_mesh,
    out_type=[
        jax.ShapeDtypeStruct((NP, H), jnp.float32),
        jax.ShapeDtypeStruct((NP, H), jnp.float32),
    ],
    scratch_types=[
        pltpu.VMEM((QC, C), jnp.int32),
        pltpu.VMEM((QC, C), jnp.int32),
        pltpu.VMEM((C, H), jnp.float32),
        pltpu.VMEM((C, H), jnp.float32),
        pltpu.VMEM_SHARED((NP, H), jnp.float32),
        pltpu.SemaphoreType.DMA,
        pltpu.SemaphoreType.DMA,
        pltpu.SemaphoreType.DMA,
        pltpu.SemaphoreType.DMA,
    ],
)
def _adj_kernel(w0_hbm, w1_hbm, col2_hbm, row2_hbm, a0_hbm, a1_hbm,
                colv, rowv, bufa, bufb, acc, ga, gb, sa, sb):
    cid = lax.axis_index("c")
    sid = lax.axis_index("s")

    def _gather(c, buf, sem):
        @pl.when(cid == 0)
        def _():
            pltpu.async_copy(w0_hbm.at[colv.at[c]], buf, sem)

        @pl.when(cid == 1)
        def _():
            pltpu.async_copy(w1_hbm.at[colv.at[c]], buf, sem)

    def _gwait(c, buf, sem):
        pltpu.make_async_copy(w0_hbm.at[colv.at[c]], buf, sem).wait()

    def _swait(c, buf, sem):
        pltpu.make_async_copy(buf, acc.at[rowv.at[c]], sem).wait()

    def _zrow(i, _):
        def _zc(k, _2):
            bufa[i, pl.ds(k * 16, 16)] = jnp.zeros((16,), jnp.float32)
            return 0

        lax.fori_loop(0, 8, _zc, 0)
        return 0

    lax.fori_loop(0, C, _zrow, 0)

    def _zacc(t, _):
        pltpu.sync_copy(bufa, acc.at[pl.ds(sid * RPT + t * C, C)])
        return 0

    lax.fori_loop(0, RPT // C, _zacc, 0)
    plsc.subcore_barrier()

    def _quarter(q, _):
        pltpu.sync_copy(col2_hbm.at[pl.ds(sid * CPS + q * QC, QC)], colv)
        pltpu.sync_copy(row2_hbm.at[pl.ds(sid * CPS + q * QC, QC)], rowv)
        _gather(0, bufa, ga)
        _gather(1, bufb, gb)

        def _step(jj, _):
            c0 = 2 * jj
            c1 = c0 + 1
            _gwait(c0, bufa, ga)
            pltpu.async_copy(bufa, acc.at[rowv.at[c0]], sa, add=True)
            _gwait(c1, bufb, gb)
            pltpu.async_copy(bufb, acc.at[rowv.at[c1]], sb, add=True)

            @pl.when(jj < QC // 2 - 1)
            def _():
                _swait(c0, bufa, sa)
                _gather(c0 + 2, bufa, ga)
                _swait(c1, bufb, sb)
                _gather(c1 + 2, bufb, gb)

            return 0

        lax.fori_loop(0, QC // 2, _step, 0)
        _swait(QC - 2, bufa, sa)
        _swait(QC - 1, bufb, sb)
        return 0

    lax.fori_loop(0, CPS // QC, _quarter, 0)
    plsc.subcore_barrier()

    def _wout(t, _):
        r0 = sid * RPT + t * C
        pltpu.sync_copy(acc.at[pl.ds(r0, C)], bufa)

        @pl.when(cid == 0)
        def _():
            pltpu.sync_copy(bufa, a0_hbm.at[pl.ds(r0, C)])

        @pl.when(cid == 1)
        def _():
            pltpu.sync_copy(bufa, a1_hbm.at[pl.ds(r0, C)])

        return 0

    lax.fori_loop(0, RPT // C, _wout, 0)


def _mm_body(x_ref, wt_ref, b_ref, o_ref):
    o_ref[...] = (
        jnp.dot(x_ref[...], wt_ref[...], preferred_element_type=jnp.float32)
        + b_ref[...]
    )


def _matmul(xp, wt, b2):
    return pl.pallas_call(
        _mm_body,
        grid=(NP // 256,),
        in_specs=[
            pl.BlockSpec((256, D), lambda i: (i, 0)),
            pl.BlockSpec((D, D), lambda i: (0, 0)),
            pl.BlockSpec((1, D), lambda i: (0, 0)),
        ],
        out_specs=pl.BlockSpec((256, D), lambda i: (i, 0)),
        out_shape=jax.ShapeDtypeStruct((NP, D), jnp.float32),
    )(xp, wt, b2)


def _dinv_from(deg0_ref, deg1_ref):
    deg = deg0_ref[:, 0:1] + deg1_ref[:, 0:1]
    return jnp.where(deg > 0.0, lax.rsqrt(jnp.maximum(deg, 1e-30)), 0.0)


def _scale0_body(deg0_ref, deg1_ref, z_ref, w0_ref, w1_ref):
    dinv = _dinv_from(deg0_ref, deg1_ref)
    w0_ref[...] = z_ref[:, :H] * dinv
    w1_ref[...] = z_ref[:, H:] * dinv


def _scale0(deg0, deg1, z):
    return pl.pallas_call(
        _scale0_body,
        grid=(NP // 512,),
        in_specs=[
            pl.BlockSpec((512, H), lambda i: (i, 0)),
            pl.BlockSpec((512, H), lambda i: (i, 0)),
            pl.BlockSpec((512, D), lambda i: (i, 0)),
        ],
        out_specs=[
            pl.BlockSpec((512, H), lambda i: (i, 0)),
            pl.BlockSpec((512, H), lambda i: (i, 0)),
        ],
        out_shape=[
            jax.ShapeDtypeStruct((NP, H), jnp.float32),
            jax.ShapeDtypeStruct((NP, H), jnp.float32),
        ],
    )(deg0, deg1, z)


def _combine_body(c_ref, deg0_ref, deg1_ref, a0_ref, a1_ref, o_ref,
                  out_ref, wn0_ref, wn1_ref):
    dinv = _dinv_from(deg0_ref, deg1_ref)
    c = c_ref[0]
    t0 = a0_ref[...] * dinv
    t1 = a1_ref[...] * dinv
    out_ref[:, :H] = o_ref[:, :H] + c * t0
    out_ref[:, H:] = o_ref[:, H:] + c * t1
    wn0_ref[...] = t0 * dinv
    wn1_ref[...] = t1 * dinv


def _combine(coef, deg0, deg1, a0, a1, o):
    return pl.pallas_call(
        _combine_body,
        grid=(NP // 512,),
        in_specs=[
            pl.BlockSpec(memory_space=pltpu.SMEM),
            pl.BlockSpec((512, H), lambda i: (i, 0)),
            pl.BlockSpec((512, H), lambda i: (i, 0)),
            pl.BlockSpec((512, H), lambda i: (i, 0)),
            pl.BlockSpec((512, H), lambda i: (i, 0)),
            pl.BlockSpec((512, D), lambda i: (i, 0)),
        ],
        out_specs=[
            pl.BlockSpec((512, D), lambda i: (i, 0)),
            pl.BlockSpec((512, H), lambda i: (i, 0)),
            pl.BlockSpec((512, H), lambda i: (i, 0)),
        ],
        out_shape=[
            jax.ShapeDtypeStruct((NP, D), jnp.float32),
            jax.ShapeDtypeStruct((NP, H), jnp.float32),
            jax.ShapeDtypeStruct((NP, H), jnp.float32),
        ],
    )(coef, deg0, deg1, a0, a1, o)


def kernel(x, edge_index, W, b, weights):
    row = edge_index[0].astype(jnp.int32)
    col = edge_index[1].astype(jnp.int32)
    pad = jnp.full((EPAD - E0,), TRASH, jnp.int32)
    row2 = jnp.concatenate([row, pad]).reshape(EROWS, C)
    col2 = jnp.concatenate([col, pad]).reshape(EROWS, C)
    xp = jnp.pad(x, ((0, NP - N0), (0, 0)))
    wt = W.T
    b2 = b.reshape(1, D)

    z = _matmul(xp, wt, b2)
    deg0, deg1 = _deg_kernel(row2)
    w0, w1 = _scale0(deg0, deg1, z)

    c1 = weights[0].reshape(1)
    c2 = weights[1].reshape(1)
    c3 = (weights[0] * weights[1]).reshape(1)

    a0, a1 = _adj_kernel(w0, w1, col2, row2)
    o, w0, w1 = _combine(c1, deg0, deg1, a0, a1, z)
    a0, a1 = _adj_kernel(w0, w1, col2, row2)
    o, w0, w1 = _combine(c2, deg0, deg1, a0, a1, o)
    a0, a1 = _adj_kernel(w0, w1, col2, row2)
    o, _, _ = _combine(c3, deg0, deg1, a0, a1, o)
    return o[:N0]


# speedup vs baseline: 5.3134x; 1.1033x over previous
import functools

import jax
import jax.numpy as jnp
from jax import lax
from jax.experimental import pallas as pl
from jax.experimental.pallas import tpu as pltpu
from jax.experimental.pallas import tpu_sc as plsc

N0 = 10000
E0 = 160000
D = 256
H = 128
NP = 10240
TRASH = N0
EPAD = 163840
C = 128
EROWS = EPAD // C
CPT = EROWS // 32
RPT = NP // 16
CA = 64
ACH = EPAD // 16 // CA
SEGC = 32
NSEG = ACH // SEGC
GRP = SEGC // 4

_mesh = plsc.VectorSubcoreMesh(core_axis_name="c", subcore_axis_name="s")


@functools.partial(
    pl.kernel,
    mesh=_mesh,
    out_type=[
        jax.ShapeDtypeStruct((NP, H), jnp.float32),
        jax.ShapeDtypeStruct((NP, H), jnp.float32),
    ],
    scratch_types=[
        pltpu.VMEM((CPT, C), jnp.int32),
        pltpu.VMEM((C, H), jnp.float32),
        pltpu.VMEM_SHARED((NP, H), jnp.float32),
        pltpu.SemaphoreType.DMA,
    ],
)
def _deg_kernel(row2_hbm, deg0_hbm, deg1_hbm, rowv, ones_v, acc, sem):
    cid = lax.axis_index("c")
    sid = lax.axis_index("s")
    wid = cid * 16 + sid

    def _zrow(i, _):
        def _zc(k, _2):
            ones_v[i, pl.ds(k * 16, 16)] = jnp.zeros((16,), jnp.float32)
            return 0

        lax.fori_loop(0, 8, _zc, 0)
        return 0

    lax.fori_loop(0, C, _zrow, 0)

    def _zacc(t, _):
        pltpu.sync_copy(ones_v, acc.at[pl.ds(sid * RPT + t * C, C)])
        return 0

    lax.fori_loop(0, RPT // C, _zacc, 0)

    def _frow(i, _):
        def _fc(k, _2):
            ones_v[i, pl.ds(k * 16, 16)] = jnp.ones((16,), jnp.float32)
            return 0

        lax.fori_loop(0, 8, _fc, 0)
        return 0

    lax.fori_loop(0, C, _frow, 0)
    plsc.subcore_barrier()

    pltpu.sync_copy(row2_hbm.at[pl.ds(wid * CPT, CPT)], rowv)

    def _step(g, _):
        c0 = 4 * g
        d0 = pltpu.async_copy(ones_v, acc.at[rowv.at[c0]], sem, add=True)
        d1 = pltpu.async_copy(ones_v, acc.at[rowv.at[c0 + 1]], sem, add=True)
        d2 = pltpu.async_copy(ones_v, acc.at[rowv.at[c0 + 2]], sem, add=True)
        d3 = pltpu.async_copy(ones_v, acc.at[rowv.at[c0 + 3]], sem, add=True)
        d0.wait()
        d1.wait()
        d2.wait()
        d3.wait()
        return 0

    lax.fori_loop(0, CPT // 4, _step, 0)
    plsc.subcore_barrier()

    def _wout(t, _):
        r0 = sid * RPT + t * C
        pltpu.sync_copy(acc.at[pl.ds(r0, C)], ones_v)

        @pl.when(cid == 0)
        def _():
            pltpu.sync_copy(ones_v, deg0_hbm.at[pl.ds(r0, C)])

        @pl.when(cid == 1)
        def _():
            pltpu.sync_copy(ones_v, deg1_hbm.at[pl.ds(r0, C)])

        return 0

    lax.fori_loop(0, RPT // C, _wout, 0)


@functools.partial(
    pl.kernel,
    mesh=_mesh,
    out_type=[
        jax.ShapeDtypeStruct((NP, H), jnp.float32),
        jax.ShapeDtypeStruct((NP, H), jnp.float32),
    ],
    scratch_types=[
        pltpu.VMEM((SEGC, CA), jnp.int32),
        pltpu.VMEM((SEGC, CA), jnp.int32),
        pltpu.VMEM((CA, H), jnp.float32),
        pltpu.VMEM((CA, H), jnp.float32),
        pltpu.VMEM((CA, H), jnp.float32),
        pltpu.VMEM((CA, H), jnp.float32),
        pltpu.VMEM_SHARED((NP, H), jnp.float32),
        pltpu.SemaphoreType.DMA,
        pltpu.SemaphoreType.DMA,
        pltpu.SemaphoreType.DMA,
        pltpu.SemaphoreType.DMA,
        pltpu.SemaphoreType.DMA,
        pltpu.SemaphoreType.DMA,
        pltpu.SemaphoreType.DMA,
        pltpu.SemaphoreType.DMA,
    ],
)
def _adj_kernel(w0_hbm, w1_hbm, col2_hbm, row2_hbm, a0_hbm, a1_hbm,
                colv, rowv, b0, b1, b2, b3, acc,
                g0, g1, g2, g3, s0, s1, s2, s3):
    cid = lax.axis_index("c")
    sid = lax.axis_index("s")
    bufs = (b0, b1, b2, b3)
    gsem = (g0, g1, g2, g3)
    ssem = (s0, s1, s2, s3)

    def _gather(c, buf, sem):
        @pl.when(cid == 0)
        def _():
            pltpu.async_copy(w0_hbm.at[colv.at[c]], buf, sem)

        @pl.when(cid == 1)
        def _():
            pltpu.async_copy(w1_hbm.at[colv.at[c]], buf, sem)

    def _gwait(c, buf, sem):
        pltpu.make_async_copy(w0_hbm.at[colv.at[c]], buf, sem).wait()

    def _swait(c, buf, sem):
        pltpu.make_async_copy(buf, acc.at[rowv.at[c]], sem).wait()

    def _zrow(i, _):
        def _zc(k, _2):
            b0[i, pl.ds(k * 16, 16)] = jnp.zeros((16,), jnp.float32)
            return 0

        lax.fori_loop(0, 8, _zc, 0)
        return 0

    lax.fori_loop(0, CA, _zrow, 0)

    def _zacc(t, _):
        pltpu.sync_copy(b0, acc.at[pl.ds(sid * RPT + t * CA, CA)])
        return 0

    lax.fori_loop(0, RPT // CA, _zacc, 0)
    plsc.subcore_barrier()

    def _seg(s, _):
        pltpu.sync_copy(col2_hbm.at[pl.ds(sid * ACH + s * SEGC, SEGC)], colv)
        pltpu.sync_copy(row2_hbm.at[pl.ds(sid * ACH + s * SEGC, SEGC)], rowv)
        for k in range(4):
            _gather(k, bufs[k], gsem[k])

        def _grp(g, _2):
            for k in range(4):
                c = 4 * g + k
                _gwait(c, bufs[k], gsem[k])
                pltpu.async_copy(bufs[k], acc.at[rowv.at[c]], ssem[k],
                                 add=True)

                @pl.when(g < GRP - 1)
                def _():
                    _swait(c, bufs[k], ssem[k])
                    _gather(c + 4, bufs[k], gsem[k])

            return 0

        lax.fori_loop(0, GRP, _grp, 0)
        for k in range(4):
            _swait(4 * (GRP - 1) + k, bufs[k], ssem[k])
        return 0

    lax.fori_loop(0, NSEG, _seg, 0)
    plsc.subcore_barrier()

    def _wout(t, _):
        r0 = sid * RPT + t * CA
        pltpu.sync_copy(acc.at[pl.ds(r0, CA)], b0)

        @pl.when(cid == 0)
        def _():
            pltpu.sync_copy(b0, a0_hbm.at[pl.ds(r0, CA)])

        @pl.when(cid == 1)
        def _():
            pltpu.sync_copy(b0, a1_hbm.at[pl.ds(r0, CA)])

        return 0

    lax.fori_loop(0, RPT // CA, _wout, 0)


def _mm_body(x_ref, wt_ref, b_ref, o_ref):
    o_ref[...] = (
        jnp.dot(x_ref[...], wt_ref[...], preferred_element_type=jnp.float32)
        + b_ref[...]
    )


def _matmul(xp, wt, b2):
    return pl.pallas_call(
        _mm_body,
        grid=(NP // 256,),
        in_specs=[
            pl.BlockSpec((256, D), lambda i: (i, 0)),
            pl.BlockSpec((D, D), lambda i: (0, 0)),
            pl.BlockSpec((1, D), lambda i: (0, 0)),
        ],
        out_specs=pl.BlockSpec((256, D), lambda i: (i, 0)),
        out_shape=jax.ShapeDtypeStruct((NP, D), jnp.float32),
    )(xp, wt, b2)


def _dinv_from(deg0_ref, deg1_ref):
    deg = deg0_ref[:, 0:1] + deg1_ref[:, 0:1]
    return jnp.where(deg > 0.0, lax.rsqrt(jnp.maximum(deg, 1e-30)), 0.0)


def _scale0_body(deg0_ref, deg1_ref, z_ref, w0_ref, w1_ref):
    dinv = _dinv_from(deg0_ref, deg1_ref)
    w0_ref[...] = z_ref[:, :H] * dinv
    w1_ref[...] = z_ref[:, H:] * dinv


def _scale0(deg0, deg1, z):
    return pl.pallas_call(
        _scale0_body,
        grid=(NP // 512,),
        in_specs=[
            pl.BlockSpec((512, H), lambda i: (i, 0)),
            pl.BlockSpec((512, H), lambda i: (i, 0)),
            pl.BlockSpec((512, D), lambda i: (i, 0)),
        ],
        out_specs=[
            pl.BlockSpec((512, H), lambda i: (i, 0)),
            pl.BlockSpec((512, H), lambda i: (i, 0)),
        ],
        out_shape=[
            jax.ShapeDtypeStruct((NP, H), jnp.float32),
            jax.ShapeDtypeStruct((NP, H), jnp.float32),
        ],
    )(deg0, deg1, z)


def _combine_body(c_ref, deg0_ref, deg1_ref, a0_ref, a1_ref, o_ref,
                  out_ref, wn0_ref, wn1_ref):
    dinv = _dinv_from(deg0_ref, deg1_ref)
    c = c_ref[0]
    t0 = a0_ref[...] * dinv
    t1 = a1_ref[...] * dinv
    out_ref[:, :H] = o_ref[:, :H] + c * t0
    out_ref[:, H:] = o_ref[:, H:] + c * t1
    wn0_ref[...] = t0 * dinv
    wn1_ref[...] = t1 * dinv


def _combine(coef, deg0, deg1, a0, a1, o):
    return pl.pallas_call(
        _combine_body,
        grid=(NP // 512,),
        in_specs=[
            pl.BlockSpec(memory_space=pltpu.SMEM),
            pl.BlockSpec((512, H), lambda i: (i, 0)),
            pl.BlockSpec((512, H), lambda i: (i, 0)),
            pl.BlockSpec((512, H), lambda i: (i, 0)),
            pl.BlockSpec((512, H), lambda i: (i, 0)),
            pl.BlockSpec((512, D), lambda i: (i, 0)),
        ],
        out_specs=[
            pl.BlockSpec((512, D), lambda i: (i, 0)),
            pl.BlockSpec((512, H), lambda i: (i, 0)),
            pl.BlockSpec((512, H), lambda i: (i, 0)),
        ],
        out_shape=[
            jax.ShapeDtypeStruct((NP, D), jnp.float32),
            jax.ShapeDtypeStruct((NP, H), jnp.float32),
            jax.ShapeDtypeStruct((NP, H), jnp.float32),
        ],
    )(coef, deg0, deg1, a0, a1, o)


def kernel(x, edge_index, W, b, weights):
    row = edge_index[0].astype(jnp.int32)
    col = edge_index[1].astype(jnp.int32)
    pad = jnp.full((EPAD - E0,), TRASH, jnp.int32)
    rowp = jnp.concatenate([row, pad])
    colp = jnp.concatenate([col, pad])
    row2 = rowp.reshape(EROWS, C)
    row2a = rowp.reshape(EPAD // CA, CA)
    col2a = colp.reshape(EPAD // CA, CA)
    xp = jnp.pad(x, ((0, NP - N0), (0, 0)))
    wt = W.T
    b2 = b.reshape(1, D)

    z = _matmul(xp, wt, b2)
    deg0, deg1 = _deg_kernel(row2)
    w0, w1 = _scale0(deg0, deg1, z)

    c1 = weights[0].reshape(1)
    c2 = weights[1].reshape(1)
    c3 = (weights[0] * weights[1]).reshape(1)

    a0, a1 = _adj_kernel(w0, w1, col2a, row2a)
    o, w0, w1 = _combine(c1, deg0, deg1, a0, a1, z)
    a0, a1 = _adj_kernel(w0, w1, col2a, row2a)
    o, w0, w1 = _combine(c2, deg0, deg1, a0, a1, o)
    a0, a1 = _adj_kernel(w0, w1, col2a, row2a)
    o, _, _ = _combine(c3, deg0, deg1, a0, a1, o)
    return o[:N0]
